# baseline (speedup 1.0000x reference)
import functools

import jax
import jax.numpy as jnp
from jax import lax
from jax.experimental import pallas as pl
from jax.experimental.pallas import tpu as pltpu
from jax.experimental.pallas import tpu_sc as plsc

B = 16384
D = 32
NC = 2
NS = 16
NW = NC * NS
BPW = B // NW


def _sc_gather(user_ids, item_ids, user_table, item_table):
    mesh = plsc.VectorSubcoreMesh(core_axis_name="c", subcore_axis_name="s")

    @functools.partial(
        pl.kernel,
        mesh=mesh,
        compiler_params=pltpu.CompilerParams(use_tc_tiling_on_sc=False),
        out_type=(
            jax.ShapeDtypeStruct((B, D), jnp.float32),
            jax.ShapeDtypeStruct((B, D), jnp.float32),
        ),
        scratch_types=[
            pltpu.VMEM((BPW,), jnp.int32),
            pltpu.VMEM((BPW,), jnp.int32),
            pltpu.VMEM((BPW, D), jnp.float32),
            pltpu.VMEM((BPW, D), jnp.float32),
            pltpu.SemaphoreType.DMA,
            pltpu.SemaphoreType.DMA,
        ],
    )
    def k(uid_hbm, iid_hbm, utab_hbm, itab_hbm, ou_hbm, oi_hbm,
          uidx_v, iidx_v, urows_v, irows_v, sem_u, sem_i):
        wid = lax.axis_index("s") * NC + lax.axis_index("c")
        base = wid * BPW
        pltpu.sync_copy(uid_hbm.at[pl.ds(base, BPW)], uidx_v)
        pltpu.sync_copy(iid_hbm.at[pl.ds(base, BPW)], iidx_v)
        cu = pltpu.async_copy(utab_hbm.at[uidx_v], urows_v, sem_u)
        ci = pltpu.async_copy(itab_hbm.at[iidx_v], irows_v, sem_i)
        cu.wait()
        ci.wait()
        pltpu.sync_copy(urows_v, ou_hbm.at[pl.ds(base, BPW)])
        pltpu.sync_copy(irows_v, oi_hbm.at[pl.ds(base, BPW)])

    return k(user_ids, item_ids, user_table, item_table)


def _tc_head_body(u_ref, i_ref, wu_ref, wi_ref, b_ref, o_ref):
    logits = (
        jnp.dot(u_ref[...], wu_ref[...], preferred_element_type=jnp.float32)
        + jnp.dot(i_ref[...], wi_ref[...], preferred_element_type=jnp.float32)
        + b_ref[0]
    )
    o_ref[...] = jax.nn.sigmoid(logits)


def _tc_head(u_emb, i_emb, fc_w, fc_b):
    wu = fc_w[0, :D].reshape(D, 1)
    wi = fc_w[0, D:].reshape(D, 1)
    blk = 2048
    return pl.pallas_call(
        _tc_head_body,
        grid=(B // blk,),
        in_specs=[
            pl.BlockSpec((blk, D), lambda i: (i, 0)),
            pl.BlockSpec((blk, D), lambda i: (i, 0)),
            pl.BlockSpec((D, 1), lambda i: (0, 0)),
            pl.BlockSpec((D, 1), lambda i: (0, 0)),
            pl.BlockSpec(memory_space=pltpu.SMEM),
        ],
        out_specs=pl.BlockSpec((blk, 1), lambda i: (i, 0)),
        out_shape=jax.ShapeDtypeStruct((B, 1), jnp.float32),
    )(u_emb, i_emb, wu, wi, fc_b)


def kernel(user_ids, item_ids, user_table, item_table, fc_w, fc_b):
    u_emb, i_emb = _sc_gather(user_ids, item_ids, user_table, item_table)
    return _tc_head(u_emb, i_emb, fc_w, fc_b)
